# Initial kernel scaffold; baseline (speedup 1.0000x reference)
import jax
import jax.numpy as jnp
from jax.experimental import pallas as pl


def kernel(features, means, log_proportions, candidates):
    raise NotImplementedError("write your pallas kernel here")


# speedup vs baseline: 20.4848x; 20.4848x over previous
import functools

import jax
import jax.numpy as jnp
from jax import lax
from jax.experimental import pallas as pl
from jax.experimental.pallas import tpu as pltpu
from jax.experimental.pallas import tpu_sc as plsc

N = 131072
D = 64
K = 1024
C = 8
NOISE_LOG_PROP = -5.0

NC = 2
NS = 16
NW = NC * NS
L = 16

SPW = N // NW
CHUNK_SPIKES = 256
GROUPS_PER_CHUNK = CHUNK_SPIKES // L
NCHUNK = SPW // CHUNK_SPIKES
DCH = 8

_SORT_NET = (
    (0, 1), (2, 3), (4, 5), (6, 7),
    (0, 2), (1, 3), (4, 6), (5, 7),
    (1, 2), (5, 6),
    (0, 4), (1, 5), (2, 6), (3, 7),
    (2, 4), (3, 5),
    (1, 2), (3, 4), (5, 6),
)


def _sc_body(feat_hbm, cand_hbm, meansT_hbm, lp_hbm,
             mparts_hbm, nuparts_hbm, newcand_hbm, mx_hbm, se_hbm, nll_hbm,
             means_v, qbuf, fbuf, cbuf, ncbuf, mxbuf, sebuf, nllbuf,
             btab_v, nu_v):
  wid = lax.axis_index("s") * NC + lax.axis_index("c")

  lane = lax.iota(jnp.int32, L)
  lane8 = lane * C
  lane64 = lane * D
  zero = jnp.zeros((L,), jnp.float32)

  pltpu.sync_copy(meansT_hbm, means_v)
  pltpu.sync_copy(lp_hbm, btab_v)

  def zero_nu(i, _):
    nu_v[pl.ds(i * L, L)] = zero
    return 0
  lax.fori_loop(0, K // L, zero_nu, 0)

  def chunk_a(ch, _):
    base = wid * SPW + ch * CHUNK_SPIKES
    pltpu.sync_copy(feat_hbm.at[pl.ds(base * D, CHUNK_SPIKES * D)], fbuf)
    pltpu.sync_copy(cand_hbm.at[pl.ds(base * C, CHUNK_SPIKES * C)], cbuf)

    def group_a(g, _):
      coff = g * (L * C)
      cands = [plsc.load_gather(cbuf, [lane8 + (coff + c)]) for c in range(C)]

      def dloop(dc, carry):
        accs = list(carry[:C])
        nacc = carry[C]
        d0 = dc * DCH
        fs = [plsc.load_gather(fbuf, [lane64 + (g * (L * D) + d0 + dd)])
              for dd in range(DCH)]
        for dd in range(DCH):
          nacc = nacc + fs[dd] * fs[dd]
        for c in range(C):
          acc = accs[c]
          for dd in range(DCH):
            mu = plsc.load_gather(
                means_v.at[pl.ds((d0 + dd) * K, K)], [cands[c]])
            t = fs[dd] - mu
            acc = acc + t * t
          accs[c] = acc
        return tuple(accs) + (nacc,)

      out = lax.fori_loop(0, D // DCH, dloop, tuple([zero] * C) + (zero,))
      accs = out[:C]
      nacc = out[C]

      lps = [plsc.load_gather(btab_v, [cands[c]]) for c in range(C)]
      lls = [lps[c] - 0.5 * accs[c] for c in range(C)]
      nll = -0.5 * nacc + NOISE_LOG_PROP

      mx = nll
      for c in range(C):
        mx = jnp.maximum(mx, lls[c])
      es = [jnp.exp(lls[c] - mx) for c in range(C)]
      se = jnp.exp(nll - mx)
      for c in range(C):
        se = se + es[c]
      r = 1.0 / se
      qs = [es[c] * r for c in range(C)]

      soff = g * L
      mxbuf[pl.ds(soff, L)] = mx
      sebuf[pl.ds(soff, L)] = se
      nllbuf[pl.ds(soff, L)] = nll

      goff = (ch * GROUPS_PER_CHUNK + g) * (L * C)
      for c in range(C):
        plsc.addupdate_scatter(nu_v, [cands[c]], qs[c])
        qbuf[pl.ds(goff + c * L, L)] = qs[c]

      keys = list(lls)
      vals = list(cands)
      for (i, j) in _SORT_NET:
        m = keys[i] >= keys[j]
        ki = jnp.where(m, keys[i], keys[j])
        kj = jnp.where(m, keys[j], keys[i])
        vi = jnp.where(m, vals[i], vals[j])
        vj = jnp.where(m, vals[j], vals[i])
        keys[i], keys[j] = ki, kj
        vals[i], vals[j] = vi, vj
      for c in range(C):
        plsc.store_scatter(ncbuf, [lane8 + (coff + c)], vals[c])
      return 0

    lax.fori_loop(0, GROUPS_PER_CHUNK, group_a, 0)

    pltpu.sync_copy(ncbuf, newcand_hbm.at[pl.ds(base * C, CHUNK_SPIKES * C)])
    pltpu.sync_copy(mxbuf, mx_hbm.at[pl.ds(base, CHUNK_SPIKES)])
    pltpu.sync_copy(sebuf, se_hbm.at[pl.ds(base, CHUNK_SPIKES)])
    pltpu.sync_copy(nllbuf, nll_hbm.at[pl.ds(base, CHUNK_SPIKES)])
    return 0

  lax.fori_loop(0, NCHUNK, chunk_a, 0)

  def zero_m(i, _):
    means_v[pl.ds(i * L, L)] = zero
    return 0
  lax.fori_loop(0, (K * D) // L, zero_m, 0)

  def chunk_b(ch, _):
    base = wid * SPW + ch * CHUNK_SPIKES
    pltpu.sync_copy(feat_hbm.at[pl.ds(base * D, CHUNK_SPIKES * D)], fbuf)
    pltpu.sync_copy(cand_hbm.at[pl.ds(base * C, CHUNK_SPIKES * C)], cbuf)

    def group_b(g, _):
      coff = g * (L * C)
      cands = [plsc.load_gather(cbuf, [lane8 + (coff + c)]) for c in range(C)]
      goff = (ch * GROUPS_PER_CHUNK + g) * (L * C)
      qs = [qbuf[pl.ds(goff + c * L, L)] for c in range(C)]

      def dloop(dc, _):
        d0 = dc * DCH
        fs = [plsc.load_gather(fbuf, [lane64 + (g * (L * D) + d0 + dd)])
              for dd in range(DCH)]
        for c in range(C):
          for dd in range(DCH):
            plsc.addupdate_scatter(
                means_v.at[pl.ds((d0 + dd) * K, K)], [cands[c]],
                qs[c] * fs[dd])
        return 0

      lax.fori_loop(0, D // DCH, dloop, 0)
      return 0

    lax.fori_loop(0, GROUPS_PER_CHUNK, group_b, 0)
    return 0

  lax.fori_loop(0, NCHUNK, chunk_b, 0)

  pltpu.sync_copy(means_v, mparts_hbm.at[pl.ds(wid * (K * D), K * D)])
  pltpu.sync_copy(nu_v, nuparts_hbm.at[pl.ds(wid * K, K)])


_sc_kernel = functools.partial(
    pl.kernel,
    out_type=[
        jax.ShapeDtypeStruct((NW * K * D,), jnp.float32),
        jax.ShapeDtypeStruct((NW * K,), jnp.float32),
        jax.ShapeDtypeStruct((N * C,), jnp.int32),
        jax.ShapeDtypeStruct((N,), jnp.float32),
        jax.ShapeDtypeStruct((N,), jnp.float32),
        jax.ShapeDtypeStruct((N,), jnp.float32),
    ],
    mesh=plsc.VectorSubcoreMesh(
        core_axis_name="c", subcore_axis_name="s",
        num_cores=NC, num_subcores=NS),
    compiler_params=pltpu.CompilerParams(needs_layout_passes=False),
    scratch_types=[
        pltpu.VMEM((K * D,), jnp.float32),
        pltpu.VMEM((SPW * C,), jnp.float32),
        pltpu.VMEM((CHUNK_SPIKES * D,), jnp.float32),
        pltpu.VMEM((CHUNK_SPIKES * C,), jnp.int32),
        pltpu.VMEM((CHUNK_SPIKES * C,), jnp.int32),
        pltpu.VMEM((CHUNK_SPIKES,), jnp.float32),
        pltpu.VMEM((CHUNK_SPIKES,), jnp.float32),
        pltpu.VMEM((CHUNK_SPIKES,), jnp.float32),
        pltpu.VMEM((K,), jnp.float32),
        pltpu.VMEM((K,), jnp.float32),
    ],
)(_sc_body)


def _tc_finalize(mparts_ref, nuparts_ref, mx_ref, se_ref, nll_ref,
                 mdk_ref, nu_ref, elbo_ref, nn_ref):
  msum = jnp.sum(mparts_ref[...], axis=0)
  nu = jnp.sum(nuparts_ref[...], axis=0, keepdims=True)
  nu_ref[...] = nu
  mdk_ref[...] = msum / jnp.clip(nu, 1.0, None)
  mx = mx_ref[...]
  se = se_ref[...]
  logz = mx + jnp.log(se)
  elbo = jnp.sum(logz) * (1.0 / N)
  qn = jnp.exp(nll_ref[...] - mx) / se
  nn = jnp.sum(qn)
  elbo_ref[...] = jnp.full((8, 128), elbo, jnp.float32)
  nn_ref[...] = jnp.full((8, 128), nn, jnp.float32)


def kernel(features, means, log_proportions, candidates):
  meansT = means.T.reshape(-1)
  feat_flat = features.reshape(-1)
  cand_flat = candidates.reshape(-1)

  mparts, nuparts, newcand, mx, se, nll = _sc_kernel(
      feat_flat, cand_flat, meansT, log_proportions)

  mdk, nu, elbo, nn = pl.pallas_call(
      _tc_finalize,
      out_shape=[
          jax.ShapeDtypeStruct((D, K), jnp.float32),
          jax.ShapeDtypeStruct((1, K), jnp.float32),
          jax.ShapeDtypeStruct((8, 128), jnp.float32),
          jax.ShapeDtypeStruct((8, 128), jnp.float32),
      ],
  )(mparts.reshape(NW, D, K), nuparts.reshape(NW, K),
    mx.reshape(N // 128, 128), se.reshape(N // 128, 128),
    nll.reshape(N // 128, 128))

  m = mdk.T
  N_units = nu.reshape(K)
  noise_N = nn[0, 0]
  obs_elbo = elbo[0, 0]
  new_candidates = newcand.reshape(N, C)
  return m, N_units, noise_N, obs_elbo, new_candidates
